# baseline (speedup 1.0000x reference)
import jax
import jax.numpy as jnp
from jax import lax
from jax.experimental import pallas as pl
from jax.experimental.pallas import tpu as pltpu

T = 128
HIDDEN = 1024
E = 64
TOPK = 2
NBUF = 4


def _moe_body(x_ref, Wg_ref, bg_ref, b1_ref, b2_ref, W1_hbm, W2_hbm,
              out_ref, gate_ref, w1buf, w2buf, wmat_ref, sem):
    x = x_ref[...]
    logits = jnp.dot(x, Wg_ref[...], preferred_element_type=jnp.float32)
    logits = logits + bg_ref[...]
    m = jnp.max(logits, axis=1, keepdims=True)
    p = jnp.exp(logits - m)
    gate = p / jnp.sum(p, axis=1, keepdims=True)
    gate_ref[...] = gate

    iota_e = lax.broadcasted_iota(jnp.int32, (T, E), 1)
    m1 = jnp.max(gate, axis=1, keepdims=True)
    a1 = jnp.min(jnp.where(gate == m1, iota_e, E), axis=1, keepdims=True)
    sel1 = iota_e == a1
    gate2 = jnp.where(sel1, -1.0, gate)
    m2 = jnp.max(gate2, axis=1, keepdims=True)
    a2 = jnp.min(jnp.where(gate2 == m2, iota_e, E), axis=1, keepdims=True)
    sel2 = iota_e == a2
    wmat_ref[...] = jnp.where(sel1, m1, 0.0) + jnp.where(sel2, m2, 0.0)
    out_ref[...] = jnp.zeros_like(out_ref)

    def start(e):
        slot = lax.rem(e, NBUF)
        pltpu.make_async_copy(W1_hbm.at[e], w1buf.at[slot], sem.at[0, slot]).start()
        pltpu.make_async_copy(W2_hbm.at[e], w2buf.at[slot], sem.at[1, slot]).start()

    for e in range(NBUF):
        start(e)

    def step(e, _):
        slot = lax.rem(e, NBUF)
        pltpu.make_async_copy(W1_hbm.at[e], w1buf.at[slot], sem.at[0, slot]).wait()
        pltpu.make_async_copy(W2_hbm.at[e], w2buf.at[slot], sem.at[1, slot]).wait()

        onehot = (lax.broadcasted_iota(jnp.int32, (E, 1), 0) == e).astype(jnp.float32)
        col = jnp.dot(wmat_ref[...], onehot, preferred_element_type=jnp.float32)
        h = jnp.dot(x_ref[...], w1buf[slot], preferred_element_type=jnp.float32)
        h = jnp.maximum(h + b1_ref[pl.ds(e, 1), :], 0.0)
        y = jnp.dot(h, w2buf[slot], preferred_element_type=jnp.float32)
        y = y + b2_ref[pl.ds(e, 1), :]
        out_ref[...] += col * y

        @pl.when(e + NBUF < E)
        def _next():
            start(e + NBUF)

        return 0

    lax.fori_loop(0, E, step, 0)


def kernel(x, Wg, bg, W1, b1, W2, b2):
    bg2 = bg.reshape(1, E)
    out, gate = pl.pallas_call(
        _moe_body,
        in_specs=[
            pl.BlockSpec((T, HIDDEN), lambda: (0, 0)),
            pl.BlockSpec((HIDDEN, E), lambda: (0, 0)),
            pl.BlockSpec((1, E), lambda: (0, 0)),
            pl.BlockSpec((E, HIDDEN), lambda: (0, 0)),
            pl.BlockSpec((E, HIDDEN), lambda: (0, 0)),
            pl.BlockSpec(memory_space=pl.ANY),
            pl.BlockSpec(memory_space=pl.ANY),
        ],
        out_specs=[
            pl.BlockSpec((T, HIDDEN), lambda: (0, 0)),
            pl.BlockSpec((T, E), lambda: (0, 0)),
        ],
        out_shape=[
            jax.ShapeDtypeStruct((T, HIDDEN), jnp.float32),
            jax.ShapeDtypeStruct((T, E), jnp.float32),
        ],
        scratch_shapes=[
            pltpu.VMEM((NBUF, HIDDEN, HIDDEN), jnp.float32),
            pltpu.VMEM((NBUF, HIDDEN, HIDDEN), jnp.float32),
            pltpu.VMEM((T, E), jnp.float32),
            pltpu.SemaphoreType.DMA((2, NBUF)),
        ],
    )(x, Wg, bg2, b1, b2, W1, W2)
    return (out, gate)
